# Initial kernel scaffold; baseline (speedup 1.0000x reference)
import jax
import jax.numpy as jnp
from jax.experimental import pallas as pl


def kernel(x, embedding_table):
    raise NotImplementedError("write your pallas kernel here")


# speedup vs baseline: 1.1763x; 1.1763x over previous
import functools

import jax
import jax.numpy as jnp
from jax import lax
from jax.experimental import pallas as pl
from jax.experimental.pallas import tpu as pltpu
from jax.experimental.pallas import tpu_sc as plsc

NUM_EMB = 1000000
FEAT = 16
TOTAL = 16384 * 26
NC, NS, L = 2, 16, 16
NW = NC * NS
B_PER_W = TOTAL // NW
CHUNK = 1664
NCHUNK = B_PER_W // CHUNK


def _gather_body(table_hbm, idx_hbm, out_hbm, idx_v, rows_v, sem):
    wid = lax.axis_index("s") * NC + lax.axis_index("c")
    base = wid * B_PER_W
    pltpu.sync_copy(idx_hbm.at[pl.ds(base, B_PER_W)], idx_v)

    def add1(i, carry):
        for u in range(8):
            off = (i * 8 + u) * L
            idx_v[pl.ds(off, L)] = idx_v[pl.ds(off, L)] + 1
        return carry

    lax.fori_loop(0, B_PER_W // (8 * L), add1, 0)

    def chunk_body(c, carry):
        cbase = c * CHUNK
        pltpu.async_copy(
            table_hbm.at[idx_v.at[pl.ds(cbase, CHUNK)]], rows_v, sem
        ).wait()
        pltpu.sync_copy(rows_v, out_hbm.at[pl.ds(base + cbase, CHUNK)])
        return carry

    lax.fori_loop(0, NCHUNK, chunk_body, 0)


@functools.partial(jax.jit, static_argnames=())
def _launch(table, flat_idx):
    mesh = plsc.VectorSubcoreMesh(core_axis_name="c", subcore_axis_name="s")
    run = functools.partial(
        pl.kernel,
        out_type=jax.ShapeDtypeStruct((TOTAL, FEAT), jnp.float32),
        mesh=mesh,
        scratch_types=[
            pltpu.VMEM((B_PER_W,), jnp.int32),
            pltpu.VMEM((CHUNK, FEAT), jnp.float32),
            pltpu.SemaphoreType.DMA,
        ],
        compiler_params=pltpu.CompilerParams(use_tc_tiling_on_sc=False),
    )(_gather_body)
    return run(table, flat_idx)


def kernel(x, embedding_table):
    flat_idx = x.reshape(-1)
    out = _launch(embedding_table, flat_idx)
    return out.reshape(x.shape[0], x.shape[1] * FEAT)


# speedup vs baseline: 2.2569x; 1.9187x over previous
import functools

import jax
import jax.numpy as jnp
from jax import lax
from jax.experimental import pallas as pl
from jax.experimental.pallas import tpu as pltpu
from jax.experimental.pallas import tpu_sc as plsc

NUM_EMB = 1000000
FEAT = 16
TOTAL = 16384 * 26
NC, NS, L = 2, 16, 16
NW = NC * NS

W_A = 1536
MAIN = 999936
NBLK = MAIN // W_A
KMAX = (NBLK + NW - 1) // NW
TAIL = NUM_EMB - MAIN

B_PER_W = TOTAL // NW
CHUNK = 1664
NCHUNK = B_PER_W // CHUNK


def _detile_body(tableT_hbm, tail_hbm, tableL_hbm, rows_v, trans_v, tail_v):
    w = lax.axis_index("s") * NC + lax.axis_index("c")
    iota16 = lax.iota(jnp.int32, 16)
    idx_f = [iota16 * FEAT + f for f in range(FEAT)]

    def blk(k, carry):
        bi = k * NW + w

        @pl.when(bi < NBLK)
        def _():
            cb = pl.multiple_of(bi * W_A, 128)
            pltpu.sync_copy(tableT_hbm.at[:, pl.ds(cb, W_A)], rows_v)

            def grp(g, c2):
                base = g * (16 * FEAT)
                for f in range(FEAT):
                    v = rows_v[f, pl.ds(g * 16, 16)]
                    plsc.store_scatter(
                        trans_v.at[pl.ds(base, 16 * FEAT)], [idx_f[f]], v
                    )
                return c2

            lax.fori_loop(0, W_A // 16, grp, 0)
            pltpu.sync_copy(trans_v, tableL_hbm.at[pl.ds(cb * FEAT, W_A * FEAT)])

        return carry

    lax.fori_loop(0, KMAX, blk, 0)

    @pl.when(w == 0)
    def _():
        pltpu.sync_copy(tail_hbm, tail_v)
        pltpu.sync_copy(tail_v, tableL_hbm.at[pl.ds(MAIN * FEAT, TAIL * FEAT)])


def _gather_body(table_hbm, idx_hbm, out_hbm, idx_v, rows_v, sem):
    wid = lax.axis_index("s") * NC + lax.axis_index("c")
    base = wid * B_PER_W
    pltpu.sync_copy(idx_hbm.at[pl.ds(base, B_PER_W)], idx_v)

    def add1(i, carry):
        for u in range(8):
            off = (i * 8 + u) * L
            idx_v[pl.ds(off, L)] = idx_v[pl.ds(off, L)] + 1
        return carry

    lax.fori_loop(0, B_PER_W // (8 * L), add1, 0)

    def chunk_body(c, carry):
        cbase = c * CHUNK
        pltpu.async_copy(
            table_hbm.at[idx_v.at[pl.ds(cbase, CHUNK)]], rows_v, sem
        ).wait()
        pltpu.sync_copy(rows_v, out_hbm.at[pl.ds(base + cbase, CHUNK)])
        return carry

    lax.fori_loop(0, NCHUNK, chunk_body, 0)


def _mesh():
    return plsc.VectorSubcoreMesh(core_axis_name="c", subcore_axis_name="s")


@jax.jit
def _launch(table, flat_idx):
    detile = functools.partial(
        pl.kernel,
        out_type=jax.ShapeDtypeStruct((NUM_EMB * FEAT,), jnp.float32),
        mesh=_mesh(),
        scratch_types=[
            pltpu.VMEM((FEAT, W_A), jnp.float32),
            pltpu.VMEM((W_A * FEAT,), jnp.float32),
            pltpu.VMEM((TAIL * FEAT,), jnp.float32),
        ],
        compiler_params=pltpu.CompilerParams(
            use_tc_tiling_on_sc=True, needs_layout_passes=False
        ),
    )(_detile_body)
    tail = lax.slice(table, (MAIN, 0), (NUM_EMB, FEAT)).reshape(-1)
    table_lin = detile(table.T, tail).reshape(NUM_EMB, FEAT)

    gather = functools.partial(
        pl.kernel,
        out_type=jax.ShapeDtypeStruct((TOTAL, FEAT), jnp.float32),
        mesh=_mesh(),
        scratch_types=[
            pltpu.VMEM((B_PER_W,), jnp.int32),
            pltpu.VMEM((CHUNK, FEAT), jnp.float32),
            pltpu.SemaphoreType.DMA,
        ],
        compiler_params=pltpu.CompilerParams(use_tc_tiling_on_sc=False),
    )(_gather_body)
    return gather(table_lin, flat_idx)


def kernel(x, embedding_table):
    flat_idx = x.reshape(-1)
    out = _launch(embedding_table, flat_idx)
    return out.reshape(x.shape[0], x.shape[1] * FEAT)
